# baseline (speedup 1.0000x reference)
import functools

import jax
import jax.numpy as jnp
from jax import lax
from jax.experimental import pallas as pl
from jax.experimental.pallas import tpu as pltpu
from jax.experimental.pallas import tpu_sc as plsc

EPS = 1e-12
SC_CORES = 2
SC_SUBCORES = 16
NUM_TILES = SC_CORES * SC_SUBCORES


def _sc_gather(table, idx_flat):
    n = idx_flat.shape[0]
    e = table.shape[1]
    per_tile = n // NUM_TILES
    mesh = plsc.VectorSubcoreMesh(core_axis_name="c", subcore_axis_name="s")

    nchunk = 2
    ck = per_tile // nchunk

    @functools.partial(
        pl.kernel,
        mesh=mesh,
        out_type=jax.ShapeDtypeStruct((n, e), jnp.float32),
        scratch_types=[
            pltpu.VMEM((per_tile,), jnp.int32),
            pltpu.VMEM((ck, e), jnp.float32),
            pltpu.VMEM((ck, e), jnp.float32),
            pltpu.SemaphoreType.DMA,
            pltpu.SemaphoreType.DMA,
            pltpu.SemaphoreType.DMA,
        ],
    )
    def gather_kernel(table_hbm, idx_hbm, out_hbm, idx_v, buf0, buf1, gsem, w0sem, w1sem):
        wid = lax.axis_index("s") * SC_CORES + lax.axis_index("c")
        base = wid * per_tile
        pltpu.sync_copy(idx_hbm.at[pl.ds(base, per_tile)], idx_v)
        g0 = pltpu.async_copy(table_hbm.at[idx_v.at[pl.ds(0, ck)]], buf0, gsem)
        g1 = pltpu.async_copy(table_hbm.at[idx_v.at[pl.ds(ck, ck)]], buf1, gsem)
        g0.wait()
        w0 = pltpu.async_copy(buf0, out_hbm.at[pl.ds(base, ck)], w0sem)
        g1.wait()
        w1 = pltpu.async_copy(buf1, out_hbm.at[pl.ds(base + ck, ck)], w1sem)
        w0.wait()
        w1.wait()

    return gather_kernel(table, idx_flat)


def _tc_combine(gathered, seg_ids_col, pos_emb, seg_pad, ln_w, ln_b, b, s):
    e = gathered.shape[1]
    tblk = 1024
    js = s // tblk

    def body(gw_ref, sid_ref, pos_ref, seg_ref, w_ref, b_ref, o_ref):
        x = gw_ref[...]
        sid = sid_ref[...]
        seg = jnp.where(sid == 0, seg_ref[0:1, :], seg_ref[1:2, :])
        x = x + seg + pos_ref[...]
        u = jnp.mean(x, axis=-1, keepdims=True)
        v = jnp.mean((x - u) ** 2, axis=-1, keepdims=True)
        o = (x - u) * lax.rsqrt(v + EPS)
        o_ref[...] = o * w_ref[...] + b_ref[...]

    out = pl.pallas_call(
        body,
        grid=(js, b),
        in_specs=[
            pl.BlockSpec((tblk, e), lambda j, i: (i * js + j, 0)),
            pl.BlockSpec((tblk, 1), lambda j, i: (i * js + j, 0)),
            pl.BlockSpec((tblk, e), lambda j, i: (j, 0)),
            pl.BlockSpec((8, e), lambda j, i: (0, 0)),
            pl.BlockSpec((1, e), lambda j, i: (0, 0)),
            pl.BlockSpec((1, e), lambda j, i: (0, 0)),
        ],
        out_specs=pl.BlockSpec((tblk, e), lambda j, i: (i * js + j, 0)),
        out_shape=jax.ShapeDtypeStruct((b * s, e), jnp.float32),
    )(gathered, seg_ids_col, pos_emb, seg_pad, ln_w, ln_b)
    return out.reshape(b, s, e)


def kernel(token_ids, segment_ids, word_emb, seg_emb, pos_emb, ln_weight, ln_bias):
    b, s = token_ids.shape
    e = word_emb.shape[1]
    idx_flat = token_ids.astype(jnp.int32).reshape(b * s)
    seg_ids_col = segment_ids.astype(jnp.int32).reshape(b * s, 1)
    seg_pad = jnp.zeros((8, e), jnp.float32).at[: seg_emb.shape[0]].set(seg_emb)
    pos = pos_emb[:s]
    ln_w = ln_weight.reshape(1, e)
    ln_b = ln_bias.reshape(1, e)
    bh = b // 2
    nh = bh * s
    g0 = _sc_gather(word_emb, idx_flat[:nh])
    g1 = _sc_gather(word_emb, idx_flat[nh:])
    o0 = _tc_combine(g0, seg_ids_col[:nh], pos, seg_pad, ln_w, ln_b, bh, s)
    o1 = _tc_combine(g1, seg_ids_col[nh:], pos, seg_pad, ln_w, ln_b, bh, s)
    return jnp.concatenate([o0, o1], axis=0)


# speedup vs baseline: 1.2659x; 1.2659x over previous
import functools

import jax
import jax.numpy as jnp
from jax import lax
from jax.experimental import pallas as pl
from jax.experimental.pallas import tpu as pltpu
from jax.experimental.pallas import tpu_sc as plsc

EPS = 1e-12
SC_CORES = 2
SC_SUBCORES = 16
NUM_TILES = SC_CORES * SC_SUBCORES


def _sc_gather(table, idx_flat):
    n = idx_flat.shape[0]
    e = table.shape[1]
    per_tile = n // NUM_TILES
    mesh = plsc.VectorSubcoreMesh(core_axis_name="c", subcore_axis_name="s")

    nchunk = 4
    ck = per_tile // nchunk

    @functools.partial(
        pl.kernel,
        mesh=mesh,
        out_type=jax.ShapeDtypeStruct((n, e), jnp.float32),
        scratch_types=[
            pltpu.VMEM((per_tile,), jnp.int32),
            pltpu.VMEM((ck, e), jnp.float32),
            pltpu.VMEM((ck, e), jnp.float32),
            pltpu.VMEM((ck, e), jnp.float32),
            pltpu.VMEM((ck, e), jnp.float32),
            pltpu.SemaphoreType.DMA,
            pltpu.SemaphoreType.DMA,
            pltpu.SemaphoreType.DMA,
            pltpu.SemaphoreType.DMA,
            pltpu.SemaphoreType.DMA,
        ],
    )
    def gather_kernel(
        table_hbm, idx_hbm, out_hbm,
        idx_v, buf0, buf1, buf2, buf3, gsem, w0, w1, w2, w3,
    ):
        wid = lax.axis_index("s") * SC_CORES + lax.axis_index("c")
        base = wid * per_tile
        bufs = (buf0, buf1, buf2, buf3)
        wsems = (w0, w1, w2, w3)
        pltpu.sync_copy(idx_hbm.at[pl.ds(base, per_tile)], idx_v)
        gathers = [
            pltpu.async_copy(
                table_hbm.at[idx_v.at[pl.ds(k * ck, ck)]], bufs[k], gsem
            )
            for k in range(nchunk)
        ]
        writes = []
        for k in range(nchunk):
            gathers[k].wait()
            writes.append(
                pltpu.async_copy(
                    bufs[k], out_hbm.at[pl.ds(base + k * ck, ck)], wsems[k]
                )
            )
        for wcopy in writes:
            wcopy.wait()

    return gather_kernel(table, idx_flat)


def _tc_combine(gathered, seg_ids_col, pos_emb, seg_pad, ln_w, ln_b, b, s):
    e = gathered.shape[1]
    tblk = 2048
    js = s // tblk

    def body(gw_ref, sid_ref, pos_ref, seg_ref, w_ref, b_ref, o_ref):
        x = gw_ref[...]
        sid = sid_ref[...]
        seg = jnp.where(sid == 0, seg_ref[0:1, :], seg_ref[1:2, :])
        x = x + seg + pos_ref[...]
        u = jnp.mean(x, axis=-1, keepdims=True)
        u2 = jnp.mean(x * x, axis=-1, keepdims=True)
        v = u2 - u * u
        o = (x - u) * lax.rsqrt(v + EPS)
        o_ref[...] = o * w_ref[...] + b_ref[...]

    out = pl.pallas_call(
        body,
        grid=(js, b),
        in_specs=[
            pl.BlockSpec((tblk, e), lambda j, i: (i * js + j, 0)),
            pl.BlockSpec((tblk, 1), lambda j, i: (i * js + j, 0)),
            pl.BlockSpec((tblk, e), lambda j, i: (j, 0)),
            pl.BlockSpec((8, e), lambda j, i: (0, 0)),
            pl.BlockSpec((1, e), lambda j, i: (0, 0)),
            pl.BlockSpec((1, e), lambda j, i: (0, 0)),
        ],
        out_specs=pl.BlockSpec((tblk, e), lambda j, i: (i * js + j, 0)),
        out_shape=jax.ShapeDtypeStruct((b * s, e), jnp.float32),
    )(gathered, seg_ids_col, pos_emb, seg_pad, ln_w, ln_b)
    return out.reshape(b, s, e)


def kernel(token_ids, segment_ids, word_emb, seg_emb, pos_emb, ln_weight, ln_bias):
    b, s = token_ids.shape
    e = word_emb.shape[1]
    idx_flat = token_ids.astype(jnp.int32).reshape(b * s)
    gathered = _sc_gather(word_emb, idx_flat)
    seg_ids_col = segment_ids.astype(jnp.int32).reshape(b * s, 1)
    seg_pad = jnp.zeros((8, e), jnp.float32).at[: seg_emb.shape[0]].set(seg_emb)
    pos = pos_emb[:s]
    return _tc_combine(
        gathered,
        seg_ids_col,
        pos,
        seg_pad,
        ln_weight.reshape(1, e),
        ln_bias.reshape(1, e),
        b,
        s,
    )
